# baseline (speedup 1.0000x reference)
import functools

import jax
import jax.numpy as jnp
from jax import lax
from jax.experimental import pallas as pl
from jax.experimental.pallas import tpu as pltpu
from jax.experimental.pallas import tpu_sc as plsc

NUM_CLASS = 1000
N_CTX = 16
CTX_DIM = 512
SEQ_LEN = 77
SUFFIX_LEN = SEQ_LEN - 1 - N_CTX
SUF_PAD = 64
BATCH = 1024
TOK_PAD = 80
BUF_ROWS = N_CTX + SUF_PAD

NC = 2
NS = 16
NW = NC * NS
BW = BATCH // NW
NBUF = 2

_mesh = plsc.VectorSubcoreMesh(core_axis_name="c", subcore_axis_name="s",
                               num_cores=NC, num_subcores=NS)


_OUT_TYPE = [
    jax.ShapeDtypeStruct((BATCH, SEQ_LEN, CTX_DIM), jnp.float32),
    jax.ShapeDtypeStruct((BATCH, TOK_PAD), jnp.int32),
]
_SCRATCH_TYPES = [
    pltpu.VMEM((BW,), jnp.int32),
    pltpu.VMEM((BW, N_CTX), jnp.int32),
    pltpu.VMEM((BW, SUF_PAD), jnp.int32),
    pltpu.VMEM((BW, TOK_PAD), jnp.int32),
    pltpu.VMEM((BW, CTX_DIM), jnp.float32),
    pltpu.VMEM((BUF_ROWS, CTX_DIM), jnp.float32),
    pltpu.VMEM((BUF_ROWS, CTX_DIM), jnp.float32),
    pltpu.SemaphoreType.DMA,
    pltpu.SemaphoreType.DMA,
    pltpu.SemaphoreType.DMA,
    pltpu.SemaphoreType.DMA,
    pltpu.SemaphoreType.DMA,
    pltpu.SemaphoreType.DMA,
]


def _prompt_gather_body(ctx2, prefix2, suffix2, tokp, label,
                   prompts, tok_out,
                   labels_v, ctx_idx, suf_idx, tok_rows, pref_rows,
                   buf0, buf1, gsem0, gsem1, wsem0, wsem1, tsem, psem):
    wid = lax.axis_index("s") * NC + lax.axis_index("c")
    base = wid * BW
    bufs = (buf0, buf1)
    gsems = (gsem0, gsem1)
    wsems = (wsem0, wsem1)

    pltpu.sync_copy(label.at[pl.ds(base, BW)], labels_v)

    tok_cp = pltpu.make_async_copy(tokp.at[labels_v], tok_rows, tsem)
    tok_cp.start()
    pref_cp = pltpu.make_async_copy(prefix2.at[labels_v], pref_rows, psem)
    pref_cp.start()

    iota16 = lax.iota(jnp.int32, 16)
    z16 = jnp.minimum(labels_v[pl.ds(0, 16)], 0)
    for b in range(BW):
        lb = plsc.load_gather(labels_v, [z16 + b])
        ctx_idx[b, :] = lb * N_CTX + iota16
        for c in range(4):
            j = iota16 + c * 16
            if c == 3:
                j = jnp.minimum(j, SUFFIX_LEN - 1)
            suf_idx[b, pl.ds(c * 16, 16)] = lb * SUFFIX_LEN + j

    pref_cp.wait()

    def start_gather(b, k):
        buf, sem = bufs[k], gsems[k]
        pltpu.make_async_copy(
            ctx2.at[ctx_idx.at[b]],
            buf.at[pl.ds(0, N_CTX)], sem).start()
        pltpu.make_async_copy(
            suffix2.at[suf_idx.at[b]],
            buf.at[pl.ds(N_CTX, SUF_PAD)], sem).start()

    def wait_gather(k):
        pltpu.make_async_copy(ctx2.at[pl.ds(0, BUF_ROWS)], bufs[k], gsems[k]).wait()

    def start_write(b, k):
        pltpu.make_async_copy(
            pref_rows.at[pl.ds(b, 1)],
            prompts.at[base + b, pl.ds(0, 1)], wsems[k]).start()
        pltpu.make_async_copy(
            bufs[k].at[pl.ds(0, SEQ_LEN - 1)],
            prompts.at[base + b, pl.ds(1, SEQ_LEN - 1)], wsems[k]).start()

    def wait_write(b, k):
        pltpu.make_async_copy(bufs[k].at[pl.ds(0, SEQ_LEN)],
                              prompts.at[base + b], wsems[k]).wait()

    for k in range(NBUF):
        start_gather(k, k)

    def pipe_body(g, carry):
        for k in range(NBUF):
            b = g * NBUF + k
            wait_gather(k)
            start_write(b, k)
        for k in range(NBUF):
            b = g * NBUF + k
            bn = b + NBUF

            @pl.when(bn < BW)
            def _():
                wait_write(b, k)
                start_gather(bn, k)

        return carry

    lax.fori_loop(0, BW // NBUF, pipe_body, 0)

    for k in range(NBUF):
        wait_write(BW - NBUF + k, k)

    tok_cp.wait()
    tok_w = pltpu.make_async_copy(tok_rows, tok_out.at[pl.ds(base, BW)], tsem)
    tok_w.start()
    tok_w.wait()


_prompt_gather = pl.kernel(
    _prompt_gather_body,
    mesh=_mesh,
    compiler_params=pltpu.CompilerParams(use_tc_tiling_on_sc=False,
                                         needs_layout_passes=False),
    out_type=_OUT_TYPE,
    scratch_types=_SCRATCH_TYPES,
)


@jax.jit
def kernel(ctx, token_prefix, token_suffix, tokenized, label):
    ctx2 = ctx.reshape(NUM_CLASS * N_CTX, CTX_DIM)
    prefix2 = token_prefix.reshape(NUM_CLASS, CTX_DIM)
    suffix2 = token_suffix.reshape(NUM_CLASS * SUFFIX_LEN, CTX_DIM)
    tokp = jnp.pad(tokenized, ((0, 0), (0, TOK_PAD - SEQ_LEN)))
    prompts, tokg = _prompt_gather(ctx2, prefix2, suffix2, tokp,
                                   label.astype(jnp.int32))
    return prompts, tokg[:, :SEQ_LEN]
